# baseline (speedup 1.0000x reference)
import functools

import jax
import jax.numpy as jnp
from jax import lax
from jax.experimental import pallas as pl
from jax.experimental.pallas import tpu as pltpu
from jax.experimental.pallas import tpu_sc as plsc

_EMB = 64
_HID = 8

_NW = 32
_KP = 5
_CHP = _KP * 128
_SLICES = (10, 40, 50, 50, 50)


def _sc_gather_pairs(emb_table, idx2, p, row_off):
    per_w = p // _NW
    steps = per_w // _CHP
    mesh = plsc.VectorSubcoreMesh(core_axis_name="c", subcore_axis_name="s")

    @functools.partial(
        pl.kernel,
        mesh=mesh,
        out_type=jax.ShapeDtypeStruct((p, 2 * _EMB), jnp.float32),
        scratch_types=[
            pltpu.VMEM((2 * _KP, 128), jnp.int32),
            pltpu.VMEM((_KP, 128), jnp.int32),
            pltpu.VMEM((_KP, 128), jnp.int32),
            pltpu.VMEM((_CHP, _EMB), jnp.float32),
            pltpu.VMEM((_CHP, _EMB), jnp.float32),
            pltpu.SemaphoreType.DMA,
        ],
        compiler_params=pltpu.CompilerParams(use_tc_tiling_on_sc=False,
                                             needs_layout_passes=False),
    )
    def gather_kernel(table_hbm, idx_hbm, out_hbm, idxr_v, idxe_v, idxo_v,
                      re_v, ro_v, sem):
        wid = lax.axis_index("s") * 2 + lax.axis_index("c")
        base = wid * per_w
        iota = lax.broadcasted_iota(jnp.int32, (16,), 0)

        def body(i, carry):
            p0 = pl.multiple_of(base + i * _CHP, _CHP)
            r0 = pl.multiple_of(row_off + p0 // 64, 2 * _KP)
            pltpu.sync_copy(idx_hbm.at[pl.ds(r0, 2 * _KP)], idxr_v)
            for k in range(_KP * 8):
                rows16 = jnp.full((16,), k // 4, jnp.int32)
                cols = (k % 4) * 32 + 2 * iota
                ev = plsc.load_gather(idxr_v, [rows16, cols])
                od = plsc.load_gather(idxr_v, [rows16, cols + 1])
                dst = pl.ds((16 * k) % 128, 16)
                idxe_v[k // 8, dst] = ev
                idxo_v[k // 8, dst] = od
            copies = []
            for j in range(_KP):
                dst = pl.ds(j * 128, 128)
                copies.append(
                    pltpu.async_copy(table_hbm.at[idxe_v.at[j]],
                                     re_v.at[dst], sem))
                copies.append(
                    pltpu.async_copy(table_hbm.at[idxo_v.at[j]],
                                     ro_v.at[dst], sem))
            for cp in copies:
                cp.wait()
            rows = pl.ds(p0, _CHP)
            pltpu.sync_copy(re_v, out_hbm.at[rows, pl.ds(0, _EMB)])
            pltpu.sync_copy(ro_v, out_hbm.at[rows, pl.ds(_EMB, _EMB)])
            return carry

        lax.fori_loop(0, steps, body, 0)

    return gather_kernel(emb_table, idx2)


_BT = 4096


def _tc_body(t_ref, v_ref, m_ref, id_ref, g_ref, w1t, b1t, w1v, b1v, w2p,
             ms_r, out_ref, pm_ref):
    pm_ref[...] = jnp.clip(id_ref[...].astype(jnp.float32), 0.0, 1.0)
    w2 = w2p[...]
    ms = ms_r[...]
    dn = (((0,), (0,)), ((), ()))
    podd = lax.broadcasted_iota(jnp.int32, (_EMB, 128), 1) % 2 == 1
    for k in range(_BT // 128):
        row = pl.ds(k, 1)
        xt = t_ref[0, row, :]
        xv = v_ref[0, row, :]
        cm = m_ref[0, row, :].astype(jnp.float32)
        ht = jnp.tanh(w1t[...] * xt + b1t[...])
        hv = jnp.tanh(w1v[...] * xv + b1v[...]) * cm
        h = jnp.concatenate([ht, hv], axis=0)
        pr = g_ref[pl.ds(k * _EMB, _EMB), :]
        lhs = jnp.concatenate([pr, w2], axis=0)
        rhs = jnp.concatenate([ms, h], axis=0)
        res = lax.dot_general(lhs, rhs, dn,
                              preferred_element_type=jnp.float32)
        out_ref[0, :, pl.ds(k * 128, 128)] = jnp.where(
            podd, res[_EMB:, :], res[: _EMB, :])


def _tc_combine_slice(acc, t3, v3, m3, id3, gath_s, w1t, b1t, w1v, b1v, w2p,
                      ms, l, b, l0, ls):
    bt = b // _BT
    kk = _BT // 128
    specs = [
        pl.BlockSpec((1, kk, 128), lambda i, j, l0=l0: (l0 + i, j, 0)),
        pl.BlockSpec((1, kk, 128), lambda i, j, l0=l0: (l0 + i, j, 0)),
        pl.BlockSpec((1, kk, 128), lambda i, j, l0=l0: (l0 + i, j, 0)),
        pl.BlockSpec((1, kk, 128), lambda i, j, l0=l0: (l0 + i, j, 0)),
        pl.BlockSpec((_BT // 2, 128), lambda i, j, bt=bt: (i * bt + j, 0)),
        pl.BlockSpec((_HID, 1), lambda i, j: (0, 0)),
        pl.BlockSpec((_HID, 1), lambda i, j: (0, 0)),
        pl.BlockSpec((_HID, 1), lambda i, j: (0, 0)),
        pl.BlockSpec((_HID, 1), lambda i, j: (0, 0)),
        pl.BlockSpec((2 * _HID, 128), lambda i, j: (0, 0)),
        pl.BlockSpec((_EMB, 128), lambda i, j: (0, 0)),
    ]
    args = [t3, v3, m3, id3, gath_s, w1t, b1t, w1v, b1v, w2p, ms]
    kwargs = {}
    body = _tc_body
    if acc is not None:
        specs = [pl.BlockSpec((1, 8, 128), lambda i, j: (0, 0, 0)),
                 pl.BlockSpec((1, 8, 128), lambda i, j: (0, 0, 0))] + specs
        args = [acc[0], acc[1]] + args
        kwargs["input_output_aliases"] = {0: 0, 1: 1}
        body = lambda a, apm, *r: _tc_body(*r)
    return pl.pallas_call(
        body,
        grid=(ls, bt),
        in_specs=specs,
        out_specs=[
            pl.BlockSpec((1, _EMB, _BT), lambda i, j, l0=l0: (l0 + i, 0, j)),
            pl.BlockSpec((1, kk, 128), lambda i, j, l0=l0: (l0 + i, j, 0)),
        ],
        out_shape=[
            jax.ShapeDtypeStruct((l, _EMB, b), jnp.float32),
            jax.ShapeDtypeStruct((l, b // 128, 128), jnp.float32),
        ],
        **kwargs,
    )(*args)


def kernel(time, value, var_id, category_mask, W1_t, b1_t, W2_t, W1_v, b1_v,
           W2_v, emb_table):
    b, l = time.shape
    n = b * l
    p = n // 2

    id_t = var_id.T.astype(jnp.int32)
    idx2 = id_t.reshape(n // 128, 128)

    l0s = [sum(_SLICES[:s]) for s in range(len(_SLICES))]
    gaths = [
        _sc_gather_pairs(emb_table, idx2, ls * b // 2, l0 * (b // 128))
        for l0, ls in zip(l0s, _SLICES)
    ]

    cols = jnp.arange(128, dtype=jnp.int32)[None, :]
    rows = jnp.arange(_EMB, dtype=jnp.int32)[:, None]
    ms = (rows == cols // 2).astype(jnp.float32)
    w2cat = jnp.concatenate([W2_t, W2_v], axis=0)
    w2p = jnp.concatenate([w2cat, w2cat], axis=1)

    t3 = time.T.reshape(l, b // 128, 128)
    v3 = value.T.reshape(l, b // 128, 128)
    m3 = category_mask.T.reshape(l, b // 128, 128).astype(jnp.int32)
    w1tc = W1_t.reshape(_HID, 1)
    b1tc = b1_t.reshape(_HID, 1)
    w1vc = W1_v.reshape(_HID, 1)
    b1vc = b1_v.reshape(_HID, 1)

    id3 = id_t.reshape(l, b // 128, 128)
    acc = None
    for s, (l0, ls) in enumerate(zip(l0s, _SLICES)):
        acc = _tc_combine_slice(acc, t3, v3, m3, id3, gaths[s], w1tc, b1tc,
                                w1vc, b1vc, w2p, ms, l, b, l0, ls)
    out3, pm3 = acc
    return out3.transpose(2, 0, 1), pm3.reshape(l, b).T


# speedup vs baseline: 1.0127x; 1.0127x over previous
import functools

import jax
import jax.numpy as jnp
from jax import lax
from jax.experimental import pallas as pl
from jax.experimental.pallas import tpu as pltpu
from jax.experimental.pallas import tpu_sc as plsc

_EMB = 64
_HID = 8

_NW = 32
_KP = 5
_CHP = _KP * 128
_NS = 4


def _sc_gather_pairs(emb_table, idx2, p, row_off):
    per_w = p // _NW
    steps = per_w // _CHP
    mesh = plsc.VectorSubcoreMesh(core_axis_name="c", subcore_axis_name="s")

    @functools.partial(
        pl.kernel,
        mesh=mesh,
        out_type=jax.ShapeDtypeStruct((p, 2 * _EMB), jnp.float32),
        scratch_types=[
            pltpu.VMEM((2 * _KP, 128), jnp.int32),
            pltpu.VMEM((_KP, 128), jnp.int32),
            pltpu.VMEM((_KP, 128), jnp.int32),
            pltpu.VMEM((_CHP, _EMB), jnp.float32),
            pltpu.VMEM((_CHP, _EMB), jnp.float32),
            pltpu.SemaphoreType.DMA,
        ],
        compiler_params=pltpu.CompilerParams(use_tc_tiling_on_sc=False,
                                             needs_layout_passes=False),
    )
    def gather_kernel(table_hbm, idx_hbm, out_hbm, idxr_v, idxe_v, idxo_v,
                      re_v, ro_v, sem):
        wid = lax.axis_index("s") * 2 + lax.axis_index("c")
        base = wid * per_w
        iota = lax.broadcasted_iota(jnp.int32, (16,), 0)

        def body(i, carry):
            p0 = pl.multiple_of(base + i * _CHP, _CHP)
            r0 = pl.multiple_of(row_off + p0 // 64, 2 * _KP)
            pltpu.sync_copy(idx_hbm.at[pl.ds(r0, 2 * _KP)], idxr_v)
            for k in range(_KP * 8):
                rows16 = jnp.full((16,), k // 4, jnp.int32)
                cols = (k % 4) * 32 + 2 * iota
                ev = plsc.load_gather(idxr_v, [rows16, cols])
                od = plsc.load_gather(idxr_v, [rows16, cols + 1])
                dst = pl.ds((16 * k) % 128, 16)
                idxe_v[k // 8, dst] = ev
                idxo_v[k // 8, dst] = od
            copies = []
            for j in range(_KP):
                dst = pl.ds(j * 128, 128)
                copies.append(
                    pltpu.async_copy(table_hbm.at[idxe_v.at[j]],
                                     re_v.at[dst], sem))
                copies.append(
                    pltpu.async_copy(table_hbm.at[idxo_v.at[j]],
                                     ro_v.at[dst], sem))
            for cp in copies:
                cp.wait()
            rows = pl.ds(p0, _CHP)
            pltpu.sync_copy(re_v, out_hbm.at[rows, pl.ds(0, _EMB)])
            pltpu.sync_copy(ro_v, out_hbm.at[rows, pl.ds(_EMB, _EMB)])
            return carry

        lax.fori_loop(0, steps, body, 0)

    return gather_kernel(emb_table, idx2)


_BT = 4096


def _tc_body(t_ref, v_ref, m_ref, g_ref, w1t, b1t, w1v, b1v, w2p, ms_r,
             out_ref):
    w2 = w2p[...]
    ms = ms_r[...]
    dn = (((0,), (0,)), ((), ()))
    podd = lax.broadcasted_iota(jnp.int32, (_EMB, 128), 1) % 2 == 1
    for k in range(_BT // 128):
        row = pl.ds(k, 1)
        xt = t_ref[0, row, :]
        xv = v_ref[0, row, :]
        cm = m_ref[0, row, :].astype(jnp.float32)
        ht = jnp.tanh(w1t[...] * xt + b1t[...])
        hv = jnp.tanh(w1v[...] * xv + b1v[...]) * cm
        h = jnp.concatenate([ht, hv], axis=0)
        pr = g_ref[pl.ds(k * _EMB, _EMB), :]
        lhs = jnp.concatenate([pr, w2], axis=0)
        rhs = jnp.concatenate([ms, h], axis=0)
        res = lax.dot_general(lhs, rhs, dn,
                              preferred_element_type=jnp.float32)
        out_ref[0, :, pl.ds(k * 128, 128)] = jnp.where(
            podd, res[_EMB:, :], res[: _EMB, :])


def _tc_combine_slice(acc, t3, v3, m3, gath_s, w1t, b1t, w1v, b1v, w2p, ms,
                      l, b, l0, ls):
    bt = b // _BT
    kk = _BT // 128
    specs = [
        pl.BlockSpec((1, kk, 128), lambda i, j, l0=l0: (l0 + i, j, 0)),
        pl.BlockSpec((1, kk, 128), lambda i, j, l0=l0: (l0 + i, j, 0)),
        pl.BlockSpec((1, kk, 128), lambda i, j, l0=l0: (l0 + i, j, 0)),
        pl.BlockSpec((_BT // 2, 128), lambda i, j, bt=bt: (i * bt + j, 0)),
        pl.BlockSpec((_HID, 1), lambda i, j: (0, 0)),
        pl.BlockSpec((_HID, 1), lambda i, j: (0, 0)),
        pl.BlockSpec((_HID, 1), lambda i, j: (0, 0)),
        pl.BlockSpec((_HID, 1), lambda i, j: (0, 0)),
        pl.BlockSpec((2 * _HID, 128), lambda i, j: (0, 0)),
        pl.BlockSpec((_EMB, 128), lambda i, j: (0, 0)),
    ]
    args = [t3, v3, m3, gath_s, w1t, b1t, w1v, b1v, w2p, ms]
    kwargs = {}
    body = _tc_body
    if acc is not None:
        specs = [pl.BlockSpec((1, 8, 128), lambda i, j: (0, 0, 0))] + specs
        args = [acc] + args
        kwargs["input_output_aliases"] = {0: 0}
        body = lambda a, *r: _tc_body(*r)
    return pl.pallas_call(
        body,
        grid=(ls, bt),
        in_specs=specs,
        out_specs=pl.BlockSpec((1, _EMB, _BT),
                               lambda i, j, l0=l0: (l0 + i, 0, j)),
        out_shape=jax.ShapeDtypeStruct((l, _EMB, b), jnp.float32),
        **kwargs,
    )(*args)


def _pm_body(id_ref, pm_ref):
    pm_ref[...] = jnp.clip(id_ref[...].astype(jnp.float32), 0.0, 1.0)


def _pm_mask(id_t, l, b):
    return pl.pallas_call(
        _pm_body,
        grid=(l // 8,),
        in_specs=[pl.BlockSpec((8, b), lambda i: (i, 0))],
        out_specs=pl.BlockSpec((8, b), lambda i: (i, 0)),
        out_shape=jax.ShapeDtypeStruct((l, b), jnp.float32),
    )(id_t)


def kernel(time, value, var_id, category_mask, W1_t, b1_t, W2_t, W1_v, b1_v,
           W2_v, emb_table):
    b, l = time.shape
    n = b * l
    p = n // 2

    id_t = var_id.T.astype(jnp.int32)
    idx2 = id_t.reshape(n // 128, 128)

    rs = n // 128 // _NS
    gaths = [
        _sc_gather_pairs(emb_table, idx2, p // _NS, s * rs)
        for s in range(_NS)
    ]

    cols = jnp.arange(128, dtype=jnp.int32)[None, :]
    rows = jnp.arange(_EMB, dtype=jnp.int32)[:, None]
    ms = (rows == cols // 2).astype(jnp.float32)
    w2cat = jnp.concatenate([W2_t, W2_v], axis=0)
    w2p = jnp.concatenate([w2cat, w2cat], axis=1)

    t3 = time.T.reshape(l, b // 128, 128)
    v3 = value.T.reshape(l, b // 128, 128)
    m3 = category_mask.T.reshape(l, b // 128, 128).astype(jnp.int32)
    w1tc = W1_t.reshape(_HID, 1)
    b1tc = b1_t.reshape(_HID, 1)
    w1vc = W1_v.reshape(_HID, 1)
    b1vc = b1_v.reshape(_HID, 1)

    ls = l // _NS
    out3 = None
    for s in range(_NS):
        out3 = _tc_combine_slice(out3, t3, v3, m3, gaths[s], w1tc, b1tc,
                                 w1vc, b1vc, w2p, ms, l, b, s * ls, ls)
    pm = _pm_mask(id_t, l, b)
    return out3.transpose(2, 0, 1), pm.T
